# Initial kernel scaffold; baseline (speedup 1.0000x reference)
import jax
import jax.numpy as jnp
from jax.experimental import pallas as pl


def kernel(feat, edge_index, edge_w, eps):
    raise NotImplementedError("write your pallas kernel here")


# speedup vs baseline: 4.2518x; 4.2518x over previous
import functools

import jax
import jax.numpy as jnp
from jax import lax
from jax.experimental import pallas as pl
from jax.experimental.pallas import tpu as pltpu
from jax.experimental.pallas import tpu_sc as plsc

N = 10000
D = 128
DE = 16
E = 320000

NC = 2
NS = 16
DH = D // NC
EPT = E // NS
B = 80
NB = EPT // B
NBH = NB // NC
NP = 10240
RPT = NP // NS


def _sc_partial(featL, featR, src2, dst2, edge_w, zf, zw):
    mesh = plsc.VectorSubcoreMesh(
        core_axis_name="c", subcore_axis_name="s", num_cores=NC,
        num_subcores=NS)

    @functools.partial(
        pl.kernel,
        out_type=[
            jax.ShapeDtypeStruct((NC, NP, DH), jnp.float32),
            jax.ShapeDtypeStruct((NC, NP, DE), jnp.float32),
        ],
        mesh=mesh,
        compiler_params=pltpu.CompilerParams(use_tc_tiling_on_sc=False),
        scratch_types=[
            pltpu.VMEM((NB, B), jnp.int32),
            pltpu.VMEM((NB, B), jnp.int32),
            pltpu.VMEM((B, DH), jnp.float32),
            pltpu.VMEM((B, DE), jnp.float32),
            pltpu.VMEM_SHARED((NP, DH), jnp.float32),
            pltpu.VMEM_SHARED((NP, DE), jnp.float32),
            pltpu.SemaphoreType.DMA,
        ],
    )
    def k(fL_hbm, fR_hbm, src_hbm, dst_hbm, ew_hbm, zf_hbm, zw_hbm,
          pf_hbm, pw_hbm, src_v, dst_v, rows_v, ew_v, accf, accw, sem):
        cid = lax.axis_index("c")
        sid = lax.axis_index("s")
        row0 = sid * RPT

        pltpu.sync_copy(zf_hbm, accf.at[pl.ds(row0, RPT)])
        pltpu.sync_copy(zw_hbm, accw.at[pl.ds(row0, RPT)])

        pltpu.sync_copy(src_hbm.at[sid], src_v)
        pltpu.sync_copy(dst_hbm.at[sid], dst_v)
        plsc.subcore_barrier()

        def fbody(j, carry):
            @pl.when(cid == 0)
            def _():
                pltpu.async_copy(fL_hbm.at[src_v.at[j]], rows_v, sem).wait()

            @pl.when(cid == 1)
            def _():
                pltpu.async_copy(fR_hbm.at[src_v.at[j]], rows_v, sem).wait()

            pltpu.sync_copy(rows_v, accf.at[dst_v.at[j]], add=True)
            return carry

        lax.fori_loop(0, NB, fbody, 0)

        ebase = sid * EPT

        def wbody(jw, carry):
            j = cid * NBH + jw
            pltpu.sync_copy(ew_hbm.at[pl.ds(ebase + j * B, B)], ew_v)
            pltpu.sync_copy(ew_v, accw.at[dst_v.at[j]], add=True)
            return carry

        lax.fori_loop(0, NBH, wbody, 0)
        plsc.subcore_barrier()

        pltpu.sync_copy(accf.at[pl.ds(row0, RPT)],
                        pf_hbm.at[cid, pl.ds(row0, RPT)])
        pltpu.sync_copy(accw.at[pl.ds(row0, RPT)],
                        pw_hbm.at[cid, pl.ds(row0, RPT)])

    return k(featL, featR, src2, dst2, edge_w, zf, zw)


def _combine(feat, eps, pf, pw):
    R = 1000

    def body(eps_ref, feat_ref, pf_ref, pw_ref, out_ref):
        scale = 1.0 + eps_ref[0]
        p = jnp.concatenate([pf_ref[0], pf_ref[1]], axis=-1)
        f = scale * feat_ref[...] + p
        w = pw_ref[0] + pw_ref[1]
        out_ref[...] = jnp.concatenate([f, w], axis=-1)

    return pl.pallas_call(
        body,
        grid=(N // R,),
        in_specs=[
            pl.BlockSpec(memory_space=pltpu.SMEM),
            pl.BlockSpec((R, D), lambda i: (i, 0)),
            pl.BlockSpec((NC, R, DH), lambda i: (0, i, 0)),
            pl.BlockSpec((NC, R, DE), lambda i: (0, i, 0)),
        ],
        out_specs=pl.BlockSpec((R, D + DE), lambda i: (i, 0)),
        out_shape=jax.ShapeDtypeStruct((N, D + DE), jnp.float32),
    )(eps, feat, pf, pw)


def kernel(feat, edge_index, edge_w, eps):
    featL = feat[:, :DH]
    featR = feat[:, DH:]
    src2 = edge_index[0].reshape(NS, NB, B)
    dst2 = edge_index[1].reshape(NS, NB, B)
    zf = jnp.zeros((RPT, DH), jnp.float32)
    zw = jnp.zeros((RPT, DE), jnp.float32)
    pf, pw = _sc_partial(featL, featR, src2, dst2, edge_w, zf, zw)
    return _combine(feat, eps, pf, pw)


# speedup vs baseline: 7.8784x; 1.8530x over previous
import functools

import jax
import jax.numpy as jnp
from jax import lax
from jax.experimental import pallas as pl
from jax.experimental.pallas import tpu as pltpu
from jax.experimental.pallas import tpu_sc as plsc

N = 10000
D = 128
DE = 16
E = 320000

NC = 2
NS = 16
DH = D // NC
EPT = E // NS
B = 80
NB = EPT // B
NBH = NB // NC
NP = 10240
RPT = NP // NS
NBUF = 5


def _sc_partial(featL, featR, src2, dst2, edge_w, zf, zw):
    mesh = plsc.VectorSubcoreMesh(
        core_axis_name="c", subcore_axis_name="s", num_cores=NC,
        num_subcores=NS)

    @functools.partial(
        pl.kernel,
        out_type=[
            jax.ShapeDtypeStruct((NC, NP, DH), jnp.float32),
            jax.ShapeDtypeStruct((NC, NP, DE), jnp.float32),
        ],
        mesh=mesh,
        compiler_params=pltpu.CompilerParams(use_tc_tiling_on_sc=False),
        scratch_types=[
            pltpu.VMEM((NB, B), jnp.int32),
            pltpu.VMEM((NB, B), jnp.int32),
            pltpu.VMEM((NBUF, B, DH), jnp.float32),
            pltpu.VMEM((NBUF, B, DE), jnp.float32),
            pltpu.VMEM_SHARED((NP, DH), jnp.float32),
            pltpu.VMEM_SHARED((NP, DE), jnp.float32),
        ] + [pltpu.SemaphoreType.DMA] * (2 * NBUF),
    )
    def k(fL_hbm, fR_hbm, src_hbm, dst_hbm, ew_hbm, zf_hbm, zw_hbm,
          pf_hbm, pw_hbm, src_v, dst_v, rows_v, ew_v, accf, accw, *sems):
        fsem = sems[:NBUF]
        wsem = sems[NBUF:]
        cid = lax.axis_index("c")
        sid = lax.axis_index("s")
        row0 = sid * RPT

        pltpu.sync_copy(src_hbm.at[sid], src_v)
        pltpu.sync_copy(dst_hbm.at[sid], dst_v)

        pltpu.sync_copy(zf_hbm, accf.at[pl.ds(row0, RPT)])
        pltpu.sync_copy(zw_hbm, accw.at[pl.ds(row0, RPT)])

        def issue_feat(j, b):
            @pl.when(cid == 0)
            def _():
                pltpu.async_copy(fL_hbm.at[src_v.at[j]], rows_v.at[b],
                                 fsem[b])

            @pl.when(cid == 1)
            def _():
                pltpu.async_copy(fR_hbm.at[src_v.at[j]], rows_v.at[b],
                                 fsem[b])

        ebase = sid * EPT

        def issue_ew(jw, b):
            j = cid * NBH + jw
            pltpu.async_copy(ew_hbm.at[pl.ds(ebase + j * B, B)],
                             ew_v.at[b], wsem[b])

        for b in range(NBUF):
            issue_feat(b, b)
            issue_ew(b, b)
        plsc.subcore_barrier()

        def fbody(g, carry):
            for b in range(NBUF):
                j = g * NBUF + b
                pltpu.make_async_copy(fL_hbm.at[src_v.at[j]], rows_v.at[b],
                                      fsem[b]).wait()
                pltpu.sync_copy(rows_v.at[b], accf.at[dst_v.at[j]], add=True)

                @pl.when(j + NBUF < NB)
                def _():
                    issue_feat(j + NBUF, b)
            return carry

        lax.fori_loop(0, NB // NBUF, fbody, 0)

        def wbody(g, carry):
            for b in range(NBUF):
                jw = g * NBUF + b
                j = cid * NBH + jw
                pltpu.make_async_copy(
                    ew_hbm.at[pl.ds(ebase + j * B, B)], ew_v.at[b],
                    wsem[b]).wait()
                pltpu.sync_copy(ew_v.at[b], accw.at[dst_v.at[j]], add=True)

                @pl.when(jw + NBUF < NBH)
                def _():
                    issue_ew(jw + NBUF, b)
            return carry

        lax.fori_loop(0, NBH // NBUF, wbody, 0)
        plsc.subcore_barrier()

        pltpu.sync_copy(accf.at[pl.ds(row0, RPT)],
                        pf_hbm.at[cid, pl.ds(row0, RPT)])
        pltpu.sync_copy(accw.at[pl.ds(row0, RPT)],
                        pw_hbm.at[cid, pl.ds(row0, RPT)])

    return k(featL, featR, src2, dst2, edge_w, zf, zw)


def _combine(feat, eps, pf, pw):
    R = 1000

    def body(eps_ref, feat_ref, pf_ref, pw_ref, out_ref):
        scale = 1.0 + eps_ref[0]
        p = jnp.concatenate([pf_ref[0], pf_ref[1]], axis=-1)
        f = scale * feat_ref[...] + p
        w = pw_ref[0] + pw_ref[1]
        out_ref[...] = jnp.concatenate([f, w], axis=-1)

    return pl.pallas_call(
        body,
        grid=(N // R,),
        in_specs=[
            pl.BlockSpec(memory_space=pltpu.SMEM),
            pl.BlockSpec((R, D), lambda i: (i, 0)),
            pl.BlockSpec((NC, R, DH), lambda i: (0, i, 0)),
            pl.BlockSpec((NC, R, DE), lambda i: (0, i, 0)),
        ],
        out_specs=pl.BlockSpec((R, D + DE), lambda i: (i, 0)),
        out_shape=jax.ShapeDtypeStruct((N, D + DE), jnp.float32),
    )(eps, feat, pf, pw)


def kernel(feat, edge_index, edge_w, eps):
    featL = feat[:, :DH]
    featR = feat[:, DH:]
    src2 = edge_index[0].reshape(NS, NB, B)
    dst2 = edge_index[1].reshape(NS, NB, B)
    zf = jnp.zeros((RPT, DH), jnp.float32)
    zw = jnp.zeros((RPT, DE), jnp.float32)
    pf, pw = _sc_partial(featL, featR, src2, dst2, edge_w, zf, zw)
    return _combine(feat, eps, pf, pw)
